# baseline (speedup 1.0000x reference)
import functools

import jax
import jax.numpy as jnp
from jax import lax
from jax.experimental import pallas as pl
from jax.experimental.pallas import tpu as pltpu
from jax.experimental.pallas import tpu_sc as plsc

_N = 16384
_P_DIM = 64
_Q_DIM = 64
_NUM_WORKERS = 32
_RPW = _N // _NUM_WORKERS
_TC_BLOCK = 512


def _sc_gather(x0, d):
    mesh = plsc.VectorSubcoreMesh(core_axis_name="c", subcore_axis_name="s")

    @functools.partial(
        pl.kernel,
        out_type=jax.ShapeDtypeStruct((_N, 128), jnp.float32),
        mesh=mesh,
        scratch_types=[
            pltpu.VMEM((_RPW,), jnp.int32),
            pltpu.VMEM((_RPW, _P_DIM), jnp.float32),
            pltpu.SemaphoreType.DMA,
        ],
        compiler_params=pltpu.CompilerParams(use_tc_tiling_on_sc=False),
    )
    def gather_kernel(x0_hbm, d_hbm, outx_hbm, idx_d, rows_x, sem_x):
        wid = lax.axis_index("s") * 2 + lax.axis_index("c")
        base = wid * _RPW
        pltpu.sync_copy(d_hbm.at[pl.ds(base, _RPW)], idx_d)
        pltpu.async_copy(x0_hbm.at[idx_d], rows_x, sem_x).wait()
        pltpu.sync_copy(rows_x, outx_hbm.at[pl.ds(base, _RPW), pl.ds(0, _P_DIM)])

    return gather_kernel(x0, d)


def _tc_expand_body(x_ref, w_ref, v_ref, r_ref, o_ref):
    x = x_ref[:, :_P_DIM]
    wi = w_ref[...]
    v = v_ref[...]
    sv = jnp.sum(v * v, axis=1, keepdims=True)
    vn = v * lax.rsqrt(sv)
    oh = (lax.broadcasted_iota(jnp.int32, (x.shape[0], _Q_DIM), 1) == wi)
    w_rows = jnp.dot(oh.astype(jnp.float32), vn,
                     preferred_element_type=jnp.float32)
    sx = jnp.sum(x * x, axis=1, keepdims=True)
    xs = x * lax.rsqrt(sx)
    xrep = jnp.dot(xs, r_ref[...], preferred_element_type=jnp.float32)
    wtile = pltpu.repeat(w_rows, _P_DIM, axis=1)
    o_ref[...] = xrep * wtile


def _rmat():
    jm = jnp.arange(_P_DIM * _Q_DIM, dtype=jnp.int32) // _Q_DIM
    rmat = (jm[None, :] == jnp.arange(_P_DIM, dtype=jnp.int32)[:, None])
    return rmat.astype(jnp.float32)


def _tc_expand(rows_x, w, v0):
    grid = _N // _TC_BLOCK
    return pl.pallas_call(
        _tc_expand_body,
        grid=(grid,),
        in_specs=[
            pl.BlockSpec((_TC_BLOCK, 128), lambda i: (i, 0)),
            pl.BlockSpec((_TC_BLOCK, 1), lambda i: (i, 0)),
            pl.BlockSpec((_Q_DIM, _Q_DIM), lambda i: (0, 0)),
            pl.BlockSpec((_P_DIM, _P_DIM * _Q_DIM), lambda i: (0, 0)),
        ],
        out_specs=pl.BlockSpec((_TC_BLOCK, _P_DIM * _Q_DIM), lambda i: (i, 0)),
        out_shape=jax.ShapeDtypeStruct((_N, _P_DIM * _Q_DIM), jnp.float32),
        compiler_params=pltpu.CompilerParams(
            dimension_semantics=("arbitrary",),
        ),
    )(rows_x, w.reshape(_N, 1), v0, _rmat())


@jax.jit
def kernel(d, w, x0, v0):
    rows_x = _sc_gather(x0, d)
    return _tc_expand(rows_x, w, v0)


# speedup vs baseline: 1.0050x; 1.0050x over previous
import functools

import jax
import jax.numpy as jnp
from jax import lax
from jax.experimental import pallas as pl
from jax.experimental.pallas import tpu as pltpu
from jax.experimental.pallas import tpu_sc as plsc

_N = 16384
_P_DIM = 64
_Q_DIM = 64
_NUM_WORKERS = 32
_RPW = _N // _NUM_WORKERS
_TC_BLOCK = 1024


def _sc_gather(x0, d):
    mesh = plsc.VectorSubcoreMesh(core_axis_name="c", subcore_axis_name="s")

    @functools.partial(
        pl.kernel,
        out_type=jax.ShapeDtypeStruct((_N, 128), jnp.float32),
        mesh=mesh,
        scratch_types=[
            pltpu.VMEM((_RPW,), jnp.int32),
            pltpu.VMEM((_RPW, _P_DIM), jnp.float32),
            pltpu.SemaphoreType.DMA,
        ],
        compiler_params=pltpu.CompilerParams(use_tc_tiling_on_sc=False),
    )
    def gather_kernel(x0_hbm, d_hbm, outx_hbm, idx_d, rows_x, sem_x):
        wid = lax.axis_index("s") * 2 + lax.axis_index("c")
        base = wid * _RPW
        pltpu.sync_copy(d_hbm.at[pl.ds(base, _RPW)], idx_d)
        pltpu.async_copy(x0_hbm.at[idx_d], rows_x, sem_x).wait()
        pltpu.sync_copy(rows_x, outx_hbm.at[pl.ds(base, _RPW), pl.ds(0, _P_DIM)])

    return gather_kernel(x0, d)


def _tc_expand_body(x_ref, w_ref, v_ref, r_ref, o_ref):
    x = x_ref[:, :_P_DIM]
    wi = w_ref[...]
    v = v_ref[...]
    sv = jnp.sum(v * v, axis=1, keepdims=True)
    vn = v * lax.rsqrt(sv)
    oh = (lax.broadcasted_iota(jnp.int32, (x.shape[0], _Q_DIM), 1) == wi)
    w_rows = jnp.dot(oh.astype(jnp.float32), vn,
                     preferred_element_type=jnp.float32)
    sx = jnp.sum(x * x, axis=1, keepdims=True)
    xs = x * lax.rsqrt(sx)
    xrep = jnp.dot(xs, r_ref[...], preferred_element_type=jnp.float32)
    wtile = pltpu.repeat(w_rows, _P_DIM, axis=1)
    o_ref[...] = xrep * wtile


def _rmat():
    jm = jnp.arange(_P_DIM * _Q_DIM, dtype=jnp.int32) // _Q_DIM
    rmat = (jm[None, :] == jnp.arange(_P_DIM, dtype=jnp.int32)[:, None])
    return rmat.astype(jnp.float32)


def _tc_expand(rows_x, w, v0):
    grid = _N // _TC_BLOCK
    return pl.pallas_call(
        _tc_expand_body,
        grid=(grid,),
        in_specs=[
            pl.BlockSpec((_TC_BLOCK, 128), lambda i: (i, 0)),
            pl.BlockSpec((_TC_BLOCK, 1), lambda i: (i, 0)),
            pl.BlockSpec((_Q_DIM, _Q_DIM), lambda i: (0, 0)),
            pl.BlockSpec((_P_DIM, _P_DIM * _Q_DIM), lambda i: (0, 0)),
        ],
        out_specs=pl.BlockSpec((_TC_BLOCK, _P_DIM * _Q_DIM), lambda i: (i, 0)),
        out_shape=jax.ShapeDtypeStruct((_N, _P_DIM * _Q_DIM), jnp.float32),
        compiler_params=pltpu.CompilerParams(
            dimension_semantics=("arbitrary",),
        ),
    )(rows_x, w.reshape(_N, 1), v0, _rmat())


@jax.jit
def kernel(d, w, x0, v0):
    rows_x = _sc_gather(x0, d)
    return _tc_expand(rows_x, w, v0)
